# baseline (speedup 1.0000x reference)
import functools

import jax
import jax.numpy as jnp
from jax import lax
from jax.experimental import pallas as pl
from jax.experimental.pallas import tpu as pltpu
from jax.experimental.pallas import tpu_sc as plsc

NC, NS, LANES = 2, 16, 16
NW = NC * NS

_MESH = dict(core_axis_name="c", subcore_axis_name="s")
_SC_PARAMS = pltpu.CompilerParams(needs_layout_passes=False)


def _wid():
    return lax.axis_index("s") * NC + lax.axis_index("c")


def _sc_degrees(src, dst, n_pad):
    e = src.shape[0]
    epw = e // NW

    @functools.partial(
        pl.kernel,
        out_type=(jax.ShapeDtypeStruct((NW, n_pad), jnp.float32),
                  jax.ShapeDtypeStruct((NW, n_pad), jnp.float32)),
        mesh=plsc.VectorSubcoreMesh(**_MESH),
        compiler_params=_SC_PARAMS,
        scratch_types=[
            pltpu.VMEM((epw,), jnp.int32),
            pltpu.VMEM((epw,), jnp.int32),
            pltpu.VMEM((n_pad,), jnp.float32),
            pltpu.VMEM((n_pad,), jnp.float32),
            pltpu.SemaphoreType.DMA,
            pltpu.SemaphoreType.DMA,
        ],
    )
    def deg_kernel(src_hbm, dst_hbm, out_o, out_i, sbuf, dbuf, acc_o, acc_i,
                   sem_s, sem_d):
        w = _wid()
        base = w * epw
        cp_s = pltpu.async_copy(src_hbm.at[pl.ds(base, epw)], sbuf, sem_s)
        cp_d = pltpu.async_copy(dst_hbm.at[pl.ds(base, epw)], dbuf, sem_d)

        zeros = jnp.zeros((LANES,), jnp.float32)

        def zero_body(i, carry):
            acc_o[pl.ds(i * LANES, LANES)] = zeros
            acc_i[pl.ds(i * LANES, LANES)] = zeros
            return carry

        lax.fori_loop(0, n_pad // LANES, zero_body, 0)
        cp_s.wait()
        cp_d.wait()

        ones = jnp.full((LANES,), 1.0, jnp.float32)

        def body(g, carry):
            s16 = sbuf[pl.ds(g * LANES, LANES)]
            d16 = dbuf[pl.ds(g * LANES, LANES)]
            plsc.addupdate_scatter(acc_o, [s16], ones)
            plsc.addupdate_scatter(acc_i, [d16], ones)
            return carry

        lax.fori_loop(0, epw // LANES, body, 0)
        pltpu.sync_copy(acc_o, out_o.at[w])
        pltpu.sync_copy(acc_i, out_i.at[w])

    return deg_kernel(src, dst)


def _sc_aggregate(t_t, src, dst, chunk):
    f, n = t_t.shape
    e = src.shape[0]
    fs = f // NW
    k = e // chunk

    @functools.partial(
        pl.kernel,
        out_type=jax.ShapeDtypeStruct((f, n), jnp.float32),
        mesh=plsc.VectorSubcoreMesh(**_MESH),
        compiler_params=_SC_PARAMS,
        scratch_types=[
            pltpu.VMEM((fs, n), jnp.float32),
            pltpu.VMEM((fs, n), jnp.float32),
            pltpu.VMEM((chunk,), jnp.int32),
            pltpu.VMEM((chunk,), jnp.int32),
            pltpu.VMEM((chunk,), jnp.int32),
            pltpu.VMEM((chunk,), jnp.int32),
            pltpu.SemaphoreType.DMA,
            pltpu.SemaphoreType.DMA,
            pltpu.SemaphoreType.DMA,
            pltpu.SemaphoreType.DMA,
            pltpu.SemaphoreType.DMA,
        ],
    )
    def agg_kernel(t_hbm, src_hbm, dst_hbm, out_hbm, tbl, acc,
                   sb0, sb1, db0, db1, sem_t, ss0, ss1, sd0, sd1):
        w = _wid()
        cp_t = pltpu.async_copy(t_hbm.at[pl.ds(w * fs, fs)], tbl, sem_t)

        sbufs, dbufs = [sb0, sb1], [db0, db1]
        ssems, dsems = [ss0, ss1], [sd0, sd1]

        def start(ci):
            b = ci % 2
            cs = pltpu.async_copy(src_hbm.at[pl.ds(ci * chunk, chunk)],
                                  sbufs[b], ssems[b])
            cd = pltpu.async_copy(dst_hbm.at[pl.ds(ci * chunk, chunk)],
                                  dbufs[b], dsems[b])
            return cs, cd

        pend = [start(0)]
        if k > 1:
            pend.append(start(1))

        zeros = jnp.zeros((LANES,), jnp.float32)

        def zero_body(i, carry):
            for fr in range(fs):
                acc[fr, pl.ds(i * LANES, LANES)] = zeros
            return carry

        lax.fori_loop(0, n // LANES, zero_body, 0)
        cp_t.wait()

        fsplat = [jnp.full((LANES,), fr, jnp.int32) for fr in range(fs)]

        for ci in range(k):
            b = ci % 2
            cs, cd = pend[ci]
            cs.wait()
            cd.wait()
            sbuf, dbuf = sbufs[b], dbufs[b]

            @plsc.parallel_loop(0, chunk // LANES, unroll=8)
            def _(g):
                s16 = sbuf[pl.ds(g * LANES, LANES)]
                d16 = dbuf[pl.ds(g * LANES, LANES)]
                for fr in range(fs):
                    v = plsc.load_gather(tbl, [fsplat[fr], s16])
                    plsc.addupdate_scatter(acc, [fsplat[fr], d16], v)
            if ci + 2 < k:
                pend.append(start(ci + 2))

        pltpu.sync_copy(acc, out_hbm.at[pl.ds(w * fs, fs)])

    return agg_kernel(t_t, src, dst)


def _sc_cvec(src, dst, nd, n_pad):
    e = src.shape[0]
    epw = e // NW

    @functools.partial(
        pl.kernel,
        out_type=jax.ShapeDtypeStruct((NW, n_pad), jnp.float32),
        mesh=plsc.VectorSubcoreMesh(**_MESH),
        compiler_params=_SC_PARAMS,
        scratch_types=[
            pltpu.VMEM((epw,), jnp.int32),
            pltpu.VMEM((epw,), jnp.int32),
            pltpu.VMEM((n_pad,), jnp.float32),
            pltpu.VMEM((n_pad,), jnp.float32),
            pltpu.SemaphoreType.DMA,
            pltpu.SemaphoreType.DMA,
            pltpu.SemaphoreType.DMA,
        ],
    )
    def c_kernel(src_hbm, dst_hbm, nd_hbm, out_hbm, sbuf, dbuf, ndb, acc,
                 sem_s, sem_d, sem_n):
        w = _wid()
        base = w * epw
        cp_s = pltpu.async_copy(src_hbm.at[pl.ds(base, epw)], sbuf, sem_s)
        cp_d = pltpu.async_copy(dst_hbm.at[pl.ds(base, epw)], dbuf, sem_d)
        cp_n = pltpu.async_copy(nd_hbm, ndb, sem_n)

        zeros = jnp.zeros((LANES,), jnp.float32)

        def zero_body(i, carry):
            acc[pl.ds(i * LANES, LANES)] = zeros
            return carry

        lax.fori_loop(0, n_pad // LANES, zero_body, 0)
        cp_s.wait()
        cp_d.wait()
        cp_n.wait()

        def body(g, carry):
            s16 = sbuf[pl.ds(g * LANES, LANES)]
            d16 = dbuf[pl.ds(g * LANES, LANES)]
            v = plsc.load_gather(ndb, [d16])
            plsc.addupdate_scatter(acc, [s16], v)
            return carry

        lax.fori_loop(0, epw // LANES, body, 0)
        pltpu.sync_copy(acc, out_hbm.at[w])

    return c_kernel(src, dst, nd)


def _tc_norms(part_o, part_i):
    def body(po_ref, pi_ref, o_ref):
        deg_o = jnp.sum(po_ref[...], axis=0, keepdims=True)
        deg_i = jnp.sum(pi_ref[...], axis=0, keepdims=True)
        o_ref[0:1, :] = jnp.where(deg_o > 0, lax.rsqrt(deg_o), 0.0)
        o_ref[1:2, :] = jnp.where(deg_i > 0, lax.rsqrt(deg_i), 0.0)

    n_pad = part_o.shape[1]
    return pl.pallas_call(
        body, out_shape=jax.ShapeDtypeStruct((2, n_pad), jnp.float32),
    )(part_o, part_i)


def _tc_in_transform(x, w1, norms):
    n = x.shape[0]

    def body(x_ref, w_ref, n_ref, o_ref):
        y = lax.dot_general(w_ref[...], x_ref[...], (((0,), (1,)), ((), ())),
                            preferred_element_type=jnp.float32)
        o_ref[...] = y * n_ref[0:1, :n]

    return pl.pallas_call(
        body, out_shape=jax.ShapeDtypeStruct((w1.shape[1], n), jnp.float32),
    )(x, w1, norms)


def _tc_mid_transform(agg_t, norms, b_col, w_next):
    f, n = agg_t.shape

    def body(a_ref, n_ref, b_ref, w_ref, o_ref):
        h = a_ref[...] * n_ref[1:2, :n] + b_ref[...]
        h = jnp.where(h >= 0, h, 0.01 * h)
        y = lax.dot_general(w_ref[...], h, (((0,), (0,)), ((), ())),
                            preferred_element_type=jnp.float32)
        o_ref[...] = y * n_ref[0:1, :n]

    return pl.pallas_call(
        body, out_shape=jax.ShapeDtypeStruct((w_next.shape[1], n), jnp.float32),
    )(agg_t, norms, b_col, w_next)


def _tc_final(agg_t, norms, b_col, c_part, w3, b3_row):
    f, n = agg_t.shape

    def body(a_ref, n_ref, b_ref, c_ref, w_ref, b3_ref, o_ref):
        h = a_ref[...] * n_ref[1:2, :n] + b_ref[...]
        h = jnp.where(h >= 0, h, 0.01 * h)
        c = jnp.sum(c_ref[...], axis=0, keepdims=True)[:, :n]
        wv = c * n_ref[0:1, :n]
        s = jnp.sum(h * wv, axis=1, keepdims=True)
        out = lax.dot_general(s, w_ref[...], (((0,), (0,)), ((), ())),
                              preferred_element_type=jnp.float32)
        o_ref[...] = out * (1.0 / n) + b3_ref[...]

    return pl.pallas_call(
        body, out_shape=jax.ShapeDtypeStruct((1, w3.shape[1]), jnp.float32),
    )(agg_t, norms, b_col, c_part, w3, b3_row)


def kernel(in_feat, edge_index, W1, b1, W2, b2, W3, b3):
    n = in_feat.shape[0]
    src = edge_index[0].astype(jnp.int32)
    dst = edge_index[1].astype(jnp.int32)
    n_pad = ((n + 127) // 128) * 128

    part_o, part_i = _sc_degrees(src, dst, n_pad)
    norms = _tc_norms(part_o, part_i)
    c_part = _sc_cvec(src, dst, norms[1], n_pad)

    t1_t = _tc_in_transform(in_feat, W1, norms)
    agg1_t = _sc_aggregate(t1_t, src, dst, chunk=8000)
    t2_t = _tc_mid_transform(agg1_t, norms, b1.reshape(-1, 1), W2)
    agg2_t = _sc_aggregate(t2_t, src, dst, chunk=8000)
    out = _tc_final(agg2_t, norms, b2.reshape(-1, 1), c_part, W3,
                    b3.reshape(1, -1))
    return out.reshape(-1)


# speedup vs baseline: 1.0058x; 1.0058x over previous
import functools

import jax
import jax.numpy as jnp
from jax import lax
from jax.experimental import pallas as pl
from jax.experimental.pallas import tpu as pltpu
from jax.experimental.pallas import tpu_sc as plsc

NC, NS, LANES = 2, 16, 16
NW = NC * NS

_MESH = dict(core_axis_name="c", subcore_axis_name="s")
_SC_PARAMS = pltpu.CompilerParams(needs_layout_passes=False)


def _wid():
    return lax.axis_index("s") * NC + lax.axis_index("c")


def _sc_degrees(src, dst, n_pad):
    e = src.shape[0]
    epw = e // NW

    @functools.partial(
        pl.kernel,
        out_type=(jax.ShapeDtypeStruct((NW, n_pad), jnp.float32),
                  jax.ShapeDtypeStruct((NW, n_pad), jnp.float32)),
        mesh=plsc.VectorSubcoreMesh(**_MESH),
        compiler_params=_SC_PARAMS,
        scratch_types=[
            pltpu.VMEM((epw,), jnp.int32),
            pltpu.VMEM((epw,), jnp.int32),
            pltpu.VMEM((n_pad,), jnp.float32),
            pltpu.VMEM((n_pad,), jnp.float32),
            pltpu.SemaphoreType.DMA,
            pltpu.SemaphoreType.DMA,
        ],
    )
    def deg_kernel(src_hbm, dst_hbm, out_o, out_i, sbuf, dbuf, acc_o, acc_i,
                   sem_s, sem_d):
        w = _wid()
        base = w * epw
        cp_s = pltpu.async_copy(src_hbm.at[pl.ds(base, epw)], sbuf, sem_s)
        cp_d = pltpu.async_copy(dst_hbm.at[pl.ds(base, epw)], dbuf, sem_d)

        zeros = jnp.zeros((LANES,), jnp.float32)

        def zero_body(i, carry):
            acc_o[pl.ds(i * LANES, LANES)] = zeros
            acc_i[pl.ds(i * LANES, LANES)] = zeros
            return carry

        lax.fori_loop(0, n_pad // LANES, zero_body, 0)
        cp_s.wait()
        cp_d.wait()

        ones = jnp.full((LANES,), 1.0, jnp.float32)

        def body(g, carry):
            s16 = sbuf[pl.ds(g * LANES, LANES)]
            d16 = dbuf[pl.ds(g * LANES, LANES)]
            plsc.addupdate_scatter(acc_o, [s16], ones)
            plsc.addupdate_scatter(acc_i, [d16], ones)
            return carry

        lax.fori_loop(0, epw // LANES, body, 0)
        pltpu.sync_copy(acc_o, out_o.at[w])
        pltpu.sync_copy(acc_i, out_i.at[w])

    return deg_kernel(src, dst)


def _sc_aggregate(t_t, src, dst, chunk):
    f, n = t_t.shape
    e = src.shape[0]
    fs = f // NW
    k = e // chunk

    @functools.partial(
        pl.kernel,
        out_type=jax.ShapeDtypeStruct((f, n), jnp.float32),
        mesh=plsc.VectorSubcoreMesh(**_MESH),
        compiler_params=_SC_PARAMS,
        scratch_types=[
            pltpu.VMEM((fs, n), jnp.float32),
            pltpu.VMEM((fs, n), jnp.float32),
            pltpu.VMEM((chunk,), jnp.int32),
            pltpu.VMEM((chunk,), jnp.int32),
            pltpu.VMEM((chunk,), jnp.int32),
            pltpu.VMEM((chunk,), jnp.int32),
            pltpu.SemaphoreType.DMA,
            pltpu.SemaphoreType.DMA,
            pltpu.SemaphoreType.DMA,
            pltpu.SemaphoreType.DMA,
            pltpu.SemaphoreType.DMA,
        ],
    )
    def agg_kernel(t_hbm, src_hbm, dst_hbm, out_hbm, tbl, acc,
                   sb0, sb1, db0, db1, sem_t, ss0, ss1, sd0, sd1):
        w = _wid()
        cp_t = pltpu.async_copy(t_hbm.at[pl.ds(w * fs, fs)], tbl, sem_t)

        sbufs, dbufs = [sb0, sb1], [db0, db1]
        ssems, dsems = [ss0, ss1], [sd0, sd1]

        def start(ci):
            b = ci % 2
            cs = pltpu.async_copy(src_hbm.at[pl.ds(ci * chunk, chunk)],
                                  sbufs[b], ssems[b])
            cd = pltpu.async_copy(dst_hbm.at[pl.ds(ci * chunk, chunk)],
                                  dbufs[b], dsems[b])
            return cs, cd

        pend = [start(0)]
        if k > 1:
            pend.append(start(1))

        zeros = jnp.zeros((LANES,), jnp.float32)

        def zero_body(i, carry):
            for fr in range(fs):
                acc[fr, pl.ds(i * LANES, LANES)] = zeros
            return carry

        lax.fori_loop(0, n // LANES, zero_body, 0)
        cp_t.wait()

        fsplat = [jnp.full((LANES,), fr, jnp.int32) for fr in range(fs)]

        for ci in range(k):
            b = ci % 2
            cs, cd = pend[ci]
            cs.wait()
            cd.wait()
            sbuf, dbuf = sbufs[b], dbufs[b]

            @plsc.parallel_loop(0, chunk // LANES, unroll=8)
            def _(g):
                s16 = sbuf[pl.ds(g * LANES, LANES)]
                d16 = dbuf[pl.ds(g * LANES, LANES)]
                for fr in range(fs):
                    v = plsc.load_gather(tbl, [fsplat[fr], s16])
                    plsc.addupdate_scatter(acc, [fsplat[fr], d16], v)
            if ci + 2 < k:
                pend.append(start(ci + 2))

        pltpu.sync_copy(acc, out_hbm.at[pl.ds(w * fs, fs)])

    return agg_kernel(t_t, src, dst)


def _sc_cvec(src, dst, nd, n_pad):
    e = src.shape[0]
    epw = e // NW

    @functools.partial(
        pl.kernel,
        out_type=jax.ShapeDtypeStruct((NW, n_pad), jnp.float32),
        mesh=plsc.VectorSubcoreMesh(**_MESH),
        compiler_params=_SC_PARAMS,
        scratch_types=[
            pltpu.VMEM((epw,), jnp.int32),
            pltpu.VMEM((epw,), jnp.int32),
            pltpu.VMEM((n_pad,), jnp.float32),
            pltpu.VMEM((n_pad,), jnp.float32),
            pltpu.SemaphoreType.DMA,
            pltpu.SemaphoreType.DMA,
            pltpu.SemaphoreType.DMA,
        ],
    )
    def c_kernel(src_hbm, dst_hbm, nd_hbm, out_hbm, sbuf, dbuf, ndb, acc,
                 sem_s, sem_d, sem_n):
        w = _wid()
        base = w * epw
        cp_s = pltpu.async_copy(src_hbm.at[pl.ds(base, epw)], sbuf, sem_s)
        cp_d = pltpu.async_copy(dst_hbm.at[pl.ds(base, epw)], dbuf, sem_d)
        cp_n = pltpu.async_copy(nd_hbm, ndb, sem_n)

        zeros = jnp.zeros((LANES,), jnp.float32)

        def zero_body(i, carry):
            acc[pl.ds(i * LANES, LANES)] = zeros
            return carry

        lax.fori_loop(0, n_pad // LANES, zero_body, 0)
        cp_s.wait()
        cp_d.wait()
        cp_n.wait()

        def body(g, carry):
            s16 = sbuf[pl.ds(g * LANES, LANES)]
            d16 = dbuf[pl.ds(g * LANES, LANES)]
            v = plsc.load_gather(ndb, [d16])
            plsc.addupdate_scatter(acc, [s16], v)
            return carry

        lax.fori_loop(0, epw // LANES, body, 0)
        pltpu.sync_copy(acc, out_hbm.at[w])

    return c_kernel(src, dst, nd)


def _tc_norms_and_t1(x, w1, part_o, part_i):
    n = x.shape[0]
    n_pad = part_o.shape[1]

    def body(x_ref, w_ref, po_ref, pi_ref, nrm_ref, t1_ref):
        deg_o = jnp.sum(po_ref[...], axis=0, keepdims=True)
        deg_i = jnp.sum(pi_ref[...], axis=0, keepdims=True)
        ns = jnp.where(deg_o > 0, lax.rsqrt(deg_o), 0.0)
        nrm_ref[0:1, :] = ns
        nrm_ref[1:2, :] = jnp.where(deg_i > 0, lax.rsqrt(deg_i), 0.0)
        y = lax.dot_general(w_ref[...], x_ref[...], (((0,), (1,)), ((), ())),
                            preferred_element_type=jnp.float32)
        t1_ref[...] = y * ns[0:1, :n]

    return pl.pallas_call(
        body,
        out_shape=(jax.ShapeDtypeStruct((2, n_pad), jnp.float32),
                   jax.ShapeDtypeStruct((w1.shape[1], n), jnp.float32)),
    )(x, w1, part_o, part_i)


def _tc_mid_transform(agg_t, norms, b_col, w_next):
    f, n = agg_t.shape

    def body(a_ref, n_ref, b_ref, w_ref, o_ref):
        h = a_ref[...] * n_ref[1:2, :n] + b_ref[...]
        h = jnp.where(h >= 0, h, 0.01 * h)
        y = lax.dot_general(w_ref[...], h, (((0,), (0,)), ((), ())),
                            preferred_element_type=jnp.float32)
        o_ref[...] = y * n_ref[0:1, :n]

    return pl.pallas_call(
        body, out_shape=jax.ShapeDtypeStruct((w_next.shape[1], n), jnp.float32),
    )(agg_t, norms, b_col, w_next)


def _tc_final(agg_t, norms, b_col, c_part, w3, b3_row):
    f, n = agg_t.shape

    def body(a_ref, n_ref, b_ref, c_ref, w_ref, b3_ref, o_ref):
        h = a_ref[...] * n_ref[1:2, :n] + b_ref[...]
        h = jnp.where(h >= 0, h, 0.01 * h)
        c = jnp.sum(c_ref[...], axis=0, keepdims=True)[:, :n]
        wv = c * n_ref[0:1, :n]
        s = jnp.sum(h * wv, axis=1, keepdims=True)
        out = lax.dot_general(s, w_ref[...], (((0,), (0,)), ((), ())),
                              preferred_element_type=jnp.float32)
        o_ref[...] = out * (1.0 / n) + b3_ref[...]

    return pl.pallas_call(
        body, out_shape=jax.ShapeDtypeStruct((1, w3.shape[1]), jnp.float32),
    )(agg_t, norms, b_col, c_part, w3, b3_row)


def kernel(in_feat, edge_index, W1, b1, W2, b2, W3, b3):
    n = in_feat.shape[0]
    src = edge_index[0].astype(jnp.int32)
    dst = edge_index[1].astype(jnp.int32)
    n_pad = ((n + 127) // 128) * 128

    part_o, part_i = _sc_degrees(src, dst, n_pad)
    norms, t1_t = _tc_norms_and_t1(in_feat, W1, part_o, part_i)
    c_part = _sc_cvec(src, dst, norms[1], n_pad)
    agg1_t = _sc_aggregate(t1_t, src, dst, chunk=8000)
    t2_t = _tc_mid_transform(agg1_t, norms, b1.reshape(-1, 1), W2)
    agg2_t = _sc_aggregate(t2_t, src, dst, chunk=8000)
    out = _tc_final(agg2_t, norms, b2.reshape(-1, 1), c_part, W3,
                    b3.reshape(1, -1))
    return out.reshape(-1)


# speedup vs baseline: 1.0086x; 1.0027x over previous
import functools

import jax
import jax.numpy as jnp
from jax import lax
from jax.experimental import pallas as pl
from jax.experimental.pallas import tpu as pltpu
from jax.experimental.pallas import tpu_sc as plsc

NC, NS, LANES = 2, 16, 16
NW = NC * NS

_MESH = dict(core_axis_name="c", subcore_axis_name="s")
_SC_PARAMS = pltpu.CompilerParams(needs_layout_passes=False)


def _wid():
    return lax.axis_index("s") * NC + lax.axis_index("c")


def _sc_degrees(src, dst, n_pad):
    e = src.shape[0]
    epw = e // NW

    @functools.partial(
        pl.kernel,
        out_type=(jax.ShapeDtypeStruct((NW, n_pad), jnp.float32),
                  jax.ShapeDtypeStruct((NW, n_pad), jnp.float32)),
        mesh=plsc.VectorSubcoreMesh(**_MESH),
        compiler_params=_SC_PARAMS,
        scratch_types=[
            pltpu.VMEM((epw,), jnp.int32),
            pltpu.VMEM((epw,), jnp.int32),
            pltpu.VMEM((n_pad,), jnp.float32),
            pltpu.VMEM((n_pad,), jnp.float32),
            pltpu.SemaphoreType.DMA,
            pltpu.SemaphoreType.DMA,
        ],
    )
    def deg_kernel(src_hbm, dst_hbm, out_o, out_i, sbuf, dbuf, acc_o, acc_i,
                   sem_s, sem_d):
        w = _wid()
        base = w * epw
        cp_s = pltpu.async_copy(src_hbm.at[pl.ds(base, epw)], sbuf, sem_s)
        cp_d = pltpu.async_copy(dst_hbm.at[pl.ds(base, epw)], dbuf, sem_d)

        zeros = jnp.zeros((LANES,), jnp.float32)

        def zero_body(i, carry):
            acc_o[pl.ds(i * LANES, LANES)] = zeros
            acc_i[pl.ds(i * LANES, LANES)] = zeros
            return carry

        lax.fori_loop(0, n_pad // LANES, zero_body, 0)
        cp_s.wait()
        cp_d.wait()

        ones = jnp.full((LANES,), 1.0, jnp.float32)

        def body(g, carry):
            s16 = sbuf[pl.ds(g * LANES, LANES)]
            d16 = dbuf[pl.ds(g * LANES, LANES)]
            plsc.addupdate_scatter(acc_o, [s16], ones)
            plsc.addupdate_scatter(acc_i, [d16], ones)
            return carry

        lax.fori_loop(0, epw // LANES, body, 0)
        pltpu.sync_copy(acc_o, out_o.at[w])
        pltpu.sync_copy(acc_i, out_i.at[w])

    return deg_kernel(src, dst)


def _sc_aggregate(t_t, src, dst, chunk):
    f, n = t_t.shape
    e = src.shape[0]
    fs = f // NW
    k = e // chunk

    @functools.partial(
        pl.kernel,
        out_type=jax.ShapeDtypeStruct((f, n), jnp.float32),
        mesh=plsc.VectorSubcoreMesh(**_MESH),
        compiler_params=_SC_PARAMS,
        scratch_types=[
            pltpu.VMEM((fs, n), jnp.float32),
            pltpu.VMEM((fs, n), jnp.float32),
            pltpu.VMEM((chunk,), jnp.int32),
            pltpu.VMEM((chunk,), jnp.int32),
            pltpu.VMEM((chunk,), jnp.int32),
            pltpu.VMEM((chunk,), jnp.int32),
            pltpu.SemaphoreType.DMA,
            pltpu.SemaphoreType.DMA,
            pltpu.SemaphoreType.DMA,
            pltpu.SemaphoreType.DMA,
            pltpu.SemaphoreType.DMA,
        ],
    )
    def agg_kernel(t_hbm, src_hbm, dst_hbm, out_hbm, tbl, acc,
                   sb0, sb1, db0, db1, sem_t, ss0, ss1, sd0, sd1):
        w = _wid()
        cp_t = pltpu.async_copy(t_hbm.at[pl.ds(w * fs, fs)], tbl, sem_t)

        sbufs, dbufs = [sb0, sb1], [db0, db1]
        ssems, dsems = [ss0, ss1], [sd0, sd1]

        def start(ci):
            b = ci % 2
            cs = pltpu.async_copy(src_hbm.at[pl.ds(ci * chunk, chunk)],
                                  sbufs[b], ssems[b])
            cd = pltpu.async_copy(dst_hbm.at[pl.ds(ci * chunk, chunk)],
                                  dbufs[b], dsems[b])
            return cs, cd

        pend = [start(0)]
        if k > 1:
            pend.append(start(1))

        zeros = jnp.zeros((LANES,), jnp.float32)

        def zero_body(i, carry):
            for fr in range(fs):
                acc[fr, pl.ds(i * LANES, LANES)] = zeros
            return carry

        lax.fori_loop(0, n // LANES, zero_body, 0)
        cp_t.wait()

        fsplat = [jnp.full((LANES,), fr, jnp.int32) for fr in range(fs)]

        for ci in range(k):
            b = ci % 2
            cs, cd = pend[ci]
            cs.wait()
            cd.wait()
            sbuf, dbuf = sbufs[b], dbufs[b]

            @plsc.parallel_loop(0, chunk // LANES, unroll=8)
            def _(g):
                s16 = sbuf[pl.ds(g * LANES, LANES)]
                d16 = dbuf[pl.ds(g * LANES, LANES)]
                for fr in range(fs):
                    v = plsc.load_gather(tbl, [fsplat[fr], s16])
                    plsc.addupdate_scatter(acc, [fsplat[fr], d16], v)
            if ci + 2 < k:
                pend.append(start(ci + 2))

        pltpu.sync_copy(acc, out_hbm.at[pl.ds(w * fs, fs)])

    return agg_kernel(t_t, src, dst)


def _sc_cvec(src, dst, nd, n_pad):
    e = src.shape[0]
    epw = e // NW

    @functools.partial(
        pl.kernel,
        out_type=jax.ShapeDtypeStruct((NW, n_pad), jnp.float32),
        mesh=plsc.VectorSubcoreMesh(**_MESH),
        compiler_params=_SC_PARAMS,
        scratch_types=[
            pltpu.VMEM((epw,), jnp.int32),
            pltpu.VMEM((epw,), jnp.int32),
            pltpu.VMEM((n_pad,), jnp.float32),
            pltpu.VMEM((n_pad,), jnp.float32),
            pltpu.SemaphoreType.DMA,
            pltpu.SemaphoreType.DMA,
            pltpu.SemaphoreType.DMA,
        ],
    )
    def c_kernel(src_hbm, dst_hbm, nd_hbm, out_hbm, sbuf, dbuf, ndb, acc,
                 sem_s, sem_d, sem_n):
        w = _wid()
        base = w * epw
        cp_s = pltpu.async_copy(src_hbm.at[pl.ds(base, epw)], sbuf, sem_s)
        cp_d = pltpu.async_copy(dst_hbm.at[pl.ds(base, epw)], dbuf, sem_d)
        cp_n = pltpu.async_copy(nd_hbm, ndb, sem_n)

        zeros = jnp.zeros((LANES,), jnp.float32)

        def zero_body(i, carry):
            acc[pl.ds(i * LANES, LANES)] = zeros
            return carry

        lax.fori_loop(0, n_pad // LANES, zero_body, 0)
        cp_s.wait()
        cp_d.wait()
        cp_n.wait()

        def body(g, carry):
            s16 = sbuf[pl.ds(g * LANES, LANES)]
            d16 = dbuf[pl.ds(g * LANES, LANES)]
            v = plsc.load_gather(ndb, [d16])
            plsc.addupdate_scatter(acc, [s16], v)
            return carry

        lax.fori_loop(0, epw // LANES, body, 0)
        pltpu.sync_copy(acc, out_hbm.at[w])

    return c_kernel(src, dst, nd)


def _tc_norms_and_t1(x, w1, part_o, part_i):
    n = x.shape[0]
    n_pad = part_o.shape[1]

    def body(x_ref, w_ref, po_ref, pi_ref, nrm_ref, t1_ref):
        deg_o = jnp.sum(po_ref[...], axis=0, keepdims=True)
        deg_i = jnp.sum(pi_ref[...], axis=0, keepdims=True)
        ns = jnp.where(deg_o > 0, lax.rsqrt(deg_o), 0.0)
        nrm_ref[0:1, :] = ns
        nrm_ref[1:2, :] = jnp.where(deg_i > 0, lax.rsqrt(deg_i), 0.0)
        y = lax.dot_general(w_ref[...], x_ref[...], (((0,), (1,)), ((), ())),
                            preferred_element_type=jnp.float32)
        t1_ref[...] = y * ns[0:1, :n]

    return pl.pallas_call(
        body,
        out_shape=(jax.ShapeDtypeStruct((2, n_pad), jnp.float32),
                   jax.ShapeDtypeStruct((w1.shape[1], n), jnp.float32)),
    )(x, w1, part_o, part_i)


def _tc_mid_transform(agg_t, norms, b_col, w_next):
    f, n = agg_t.shape

    def body(a_ref, n_ref, b_ref, w_ref, o_ref):
        h = a_ref[...] * n_ref[1:2, :n] + b_ref[...]
        h = jnp.where(h >= 0, h, 0.01 * h)
        y = lax.dot_general(w_ref[...], h, (((0,), (0,)), ((), ())),
                            preferred_element_type=jnp.float32)
        o_ref[...] = y * n_ref[0:1, :n]

    return pl.pallas_call(
        body, out_shape=jax.ShapeDtypeStruct((w_next.shape[1], n), jnp.float32),
    )(agg_t, norms, b_col, w_next)


def _tc_final(agg_t, norms, b_col, c_part, w3, b3_row):
    f, n = agg_t.shape

    def body(a_ref, n_ref, b_ref, c_ref, w_ref, b3_ref, o_ref):
        h = a_ref[...] * n_ref[1:2, :n] + b_ref[...]
        h = jnp.where(h >= 0, h, 0.01 * h)
        c = jnp.sum(c_ref[...], axis=0, keepdims=True)[:, :n]
        wv = c * n_ref[0:1, :n]
        s = jnp.sum(h * wv, axis=1, keepdims=True)
        out = lax.dot_general(s, w_ref[...], (((0,), (0,)), ((), ())),
                              preferred_element_type=jnp.float32)
        o_ref[...] = out * (1.0 / n) + b3_ref[...]

    return pl.pallas_call(
        body, out_shape=jax.ShapeDtypeStruct((1, w3.shape[1]), jnp.float32),
    )(agg_t, norms, b_col, c_part, w3, b3_row)


def kernel(in_feat, edge_index, W1, b1, W2, b2, W3, b3):
    n = in_feat.shape[0]
    src = edge_index[0].astype(jnp.int32)
    dst = edge_index[1].astype(jnp.int32)
    n_pad = ((n + 127) // 128) * 128

    part_o, part_i = _sc_degrees(src, dst, n_pad)
    norms, t1_t = _tc_norms_and_t1(in_feat, W1, part_o, part_i)
    c_part = _sc_cvec(src, dst, norms[1], n_pad)
    agg1_t = _sc_aggregate(t1_t, src, dst, chunk=10000)
    t2_t = _tc_mid_transform(agg1_t, norms, b1.reshape(-1, 1), W2)
    agg2_t = _sc_aggregate(t2_t, src, dst, chunk=10000)
    out = _tc_final(agg2_t, norms, b2.reshape(-1, 1), c_part, W3,
                    b3.reshape(1, -1))
    return out.reshape(-1)
